# baseline (speedup 1.0000x reference)
import functools

import jax
import jax.numpy as jnp
from jax import lax
from jax.experimental import pallas as pl
from jax.experimental.pallas import tpu as pltpu
from jax.experimental.pallas import tpu_sc as plsc

_NC = 2
_NS = 16
_NW = _NC * _NS

_FCH = 5
_NBUF = 2


def _emb_kernel(table_ref, o_ref, *, F):
    t = table_ref[0:F, :]
    norm = jnp.sqrt(jnp.sum(t * t, axis=-1, keepdims=True))
    scale = jnp.where(norm > 1.0, 1.0 / jnp.maximum(norm, 1e-12), 1.0)
    t = t * scale
    r = lax.broadcasted_iota(jnp.int32, (F, F), 0)
    c = lax.broadcasted_iota(jnp.int32, (F, F), 1)
    perm = (c == (F - 1 - r)).astype(jnp.float32)
    o_ref[...] = jnp.dot(perm, t, preferred_element_type=jnp.float32)


def _make_sc_add(B, F, N, D):
    batch_elems = F * N * D
    row = N * D
    ch = _FCH * row
    bpw = B // _NW
    welems = bpw * batch_elems
    nchunk = welems // ch
    ch_per_batch = batch_elems // ch
    nvec = D // 16
    total = B * batch_elems

    mesh = plsc.VectorSubcoreMesh(core_axis_name="c", subcore_axis_name="s")

    @functools.partial(
        pl.kernel,
        out_type=jax.ShapeDtypeStruct((total,), jnp.float32),
        mesh=mesh,
        scratch_types=[
            pltpu.VMEM((F * D,), jnp.float32),
            pltpu.VMEM((ch,), jnp.float32),
            pltpu.VMEM((ch,), jnp.float32),
            pltpu.VMEM((ch,), jnp.float32),
            pltpu.VMEM((ch,), jnp.float32),
            pltpu.SemaphoreType.DMA((_NBUF,)),
            pltpu.SemaphoreType.DMA((_NBUF,)),
        ],
    )
    def sc_add(x_hbm, emb_hbm, o_hbm, eb, i0, i1, o0, o1, insems, outsems):
        cid = lax.axis_index("c")
        sid = lax.axis_index("s")
        wid = sid * _NC + cid
        base = wid * welems
        inb = [i0, i1]
        outb = [o0, o1]

        pltpu.sync_copy(emb_hbm, eb)
        for b in range(_NBUF):
            pltpu.async_copy(
                x_hbm.at[pl.ds(base + b * ch, ch)], inb[b], insems.at[b])

        @pl.loop(0, nchunk, step=_NBUF)
        def _outer(t):
            for b in range(_NBUF):
                g = t + b
                off = base + g * ch
                pltpu.make_async_copy(
                    x_hbm.at[pl.ds(off, ch)], inb[b], insems.at[b]).wait()

                @pl.when(t > 0)
                def _wait_out():
                    poff = base + (g - _NBUF) * ch
                    pltpu.make_async_copy(
                        outb[b], o_hbm.at[pl.ds(poff, ch)],
                        outsems.at[b]).wait()

                fc = lax.rem(g, ch_per_batch) * _FCH
                for fo in range(_FCH):
                    ebase = (fc + fo) * D
                    evs = [eb[pl.ds(ebase + 16 * v, 16)] for v in range(nvec)]

                    @pl.loop(0, N)
                    def _rows(n, fo=fo, evs=evs, b=b):
                        rb = fo * row + n * D
                        for v in range(nvec):
                            sl = pl.ds(rb + 16 * v, 16)
                            outb[b][sl] = inb[b][sl] + evs[v]

                pltpu.async_copy(
                    outb[b], o_hbm.at[pl.ds(off, ch)], outsems.at[b])

                ng = g + _NBUF

                @pl.when(ng < nchunk)
                def _next_in():
                    pltpu.async_copy(
                        x_hbm.at[pl.ds(base + ng * ch, ch)], inb[b],
                        insems.at[b])

        for b in range(_NBUF):
            goff = base + (nchunk - _NBUF + b) * ch
            pltpu.make_async_copy(
                outb[b], o_hbm.at[pl.ds(goff, ch)], outsems.at[b]).wait()

    return sc_add


def kernel(x, in_F, out_F, table):
    B, F, N, D = x.shape
    emb = pl.pallas_call(
        functools.partial(_emb_kernel, F=F),
        in_specs=[pl.BlockSpec((table.shape[0], D), lambda: (0, 0))],
        out_specs=pl.BlockSpec((F, D), lambda: (0, 0)),
        out_shape=jax.ShapeDtypeStruct((F, D), jnp.float32),
    )(table)
    sc_add = _make_sc_add(B, F, N, D)
    out = sc_add(x.reshape(-1), emb.reshape(-1))
    return out.reshape(B, F, N, D)
